# baseline (speedup 1.0000x reference)
import functools

import jax
import jax.numpy as jnp
import numpy as np
from jax import lax
from jax.experimental import pallas as pl
from jax.experimental.pallas import tpu as pltpu
from jax.experimental.pallas import tpu_sc as plsc

_B = 1024
_UMR = 20
_IMR = 20
_RML = 50
_D = 64
_TYPE_N = 10
_MONTH_N = 12
_NC = 2
_NS = 16
_NW = _NC * _NS
_BW = _B // _NW
_GB = 4
_ROWS = _GB * _RML
_BPB = _UMR // _GB
_NBLK = _BW * _BPB
_NBUF = 4
_IW = 3200
_IROWS = _B * _UMR * _RML // _IW
_IRW = _IROWS // _NW
_BLKR = _IW // _ROWS
_F32 = jnp.float32


def _build_sc_pool():
    mesh = plsc.VectorSubcoreMesh(core_axis_name="c", subcore_axis_name="s")
    bf16 = jnp.bfloat16
    out_type = jax.ShapeDtypeStruct((_B, _UMR, _D), bf16)
    scratch_types = [
        pltpu.VMEM((_IRW, _IW), jnp.int32),
        pltpu.VMEM((_NBUF, _ROWS, _D), bf16),
        pltpu.VMEM((_BW, _UMR, _D), bf16),
        tuple(pltpu.SemaphoreType.DMA for _ in range(_NBUF)),
    ]
    _MSK = np.int32(-65536)

    @functools.partial(
        pl.kernel, out_type=out_type, mesh=mesh, scratch_types=scratch_types,
        compiler_params=pltpu.CompilerParams(
            use_tc_tiling_on_sc=False, needs_layout_passes=False))
    def sc_pool(idx_hbm, tab_hbm,
                out_hbm,
                idx_v, rows_v, pool_v, sems):
        wid = lax.axis_index("s") * _NC + lax.axis_index("c")

        if True:
            pltpu.sync_copy(idx_hbm.at[pl.ds(wid * _IRW, _IRW)], idx_v)

            def start(blk, b):
                row = blk // _BLKR
                o = (blk % _BLKR) * _ROWS
                pltpu.make_async_copy(
                    tab_hbm.at[idx_v.at[row].at[pl.ds(o, _ROWS)]],
                    rows_v.at[b], sems[b]).start()

            def wait(b):
                pltpu.make_async_copy(
                    tab_hbm.at[idx_v.at[0].at[pl.ds(0, _ROWS)]],
                    rows_v.at[b], sems[b]).wait()

            for b in range(_NBUF):
                start(b, b)

            def block_body(t, carry):
                for b in range(_NBUF):
                    blk = _NBUF * t + b
                    wait(b)
                    rows = rows_v.at[b]
                    bat = blk // _BPB
                    r0 = (blk % _BPB) * _GB
                    for s in range(_GB):
                        base = s * _RML

                        def pair(r, rows=rows, base=base):
                            out = []
                            for c in range(2):
                                a = rows[base + r, pl.ds(c * 32, 32)]
                                bb = rows[base + r + 1, pl.ds(c * 32, 32)]
                                lo, hi = plsc.unpack(
                                    a + bb, format=plsc.PackFormat.INTERLEAVED)
                                out += [lo, hi]
                            return tuple(out)

                        acc = pair(0)

                        def row_body(t2, a):
                            p = pair(2 + 2 * t2)
                            return tuple(x + y for x, y in zip(a, p))

                        acc = lax.fori_loop(0, _RML // 2 - 1, row_body, acc)
                        for c in range(2):
                            packed = plsc.pack(
                                acc[2 * c], acc[2 * c + 1],
                                format=plsc.PackFormat.INTERLEAVED)
                            pool_v[bat, r0 + s, pl.ds(c * 32, 32)] = packed
                    nblk = blk + _NBUF

                    @pl.when(nblk < _NBLK)
                    def _():
                        start(nblk, b)
                return carry

            lax.fori_loop(0, _NBLK // _NBUF, block_body, 0)
            pltpu.sync_copy(pool_v, out_hbm.at[pl.ds(wid * _BW, _BW)])

    return sc_pool


_sc_pool = _build_sc_pool()

_BB = 256


def _idgather_body(uids_sm, iids_sm, uidtab_ref, iidtab_ref,
                   ue_ref, ie_ref, sem_u, sem_i):
    i = pl.program_id(0)

    def id_copy(ids_sm, tab_ref, out_ref, sem, j):
        rid = ids_sm[i * _BB + j]
        return pltpu.make_async_copy(
            tab_ref.at[pl.ds(rid, 1)], out_ref.at[pl.ds(j, 1)], sem)

    for j in range(_BB):
        id_copy(uids_sm, uidtab_ref, ue_ref, sem_u, j).start()
        id_copy(iids_sm, iidtab_ref, ie_ref, sem_i, j).start()
    for j in range(_BB):
        id_copy(uids_sm, uidtab_ref, ue_ref, sem_u, j).wait()
        id_copy(iids_sm, iidtab_ref, ie_ref, sem_i, j).wait()


def _idgather(uid_tab, iid_tab, uids, iids):
    grid_spec = pltpu.PrefetchScalarGridSpec(
        num_scalar_prefetch=2,
        grid=(_B // _BB,),
        in_specs=[
            pl.BlockSpec(memory_space=pl.ANY),
            pl.BlockSpec(memory_space=pl.ANY),
        ],
        out_specs=[
            pl.BlockSpec((_BB, _D), lambda i, *_: (i, 0)),
            pl.BlockSpec((_BB, _D), lambda i, *_: (i, 0)),
        ],
        scratch_shapes=[
            pltpu.SemaphoreType.DMA,
            pltpu.SemaphoreType.DMA,
        ],
    )
    return pl.pallas_call(
        _idgather_body,
        grid_spec=grid_spec,
        out_shape=[
            jax.ShapeDtypeStruct((_B, _D), _F32),
            jax.ShapeDtypeStruct((_B, _D), _F32),
        ],
    )(uids, iids, uid_tab, iid_tab)


def _dense_body(p_ref, w_ref, b_ref, tab_ref, v_ref, e_ref, f_ref, *, n):
    v = v_ref[...]
    oh = (v == lax.broadcasted_iota(jnp.int32, (_BB, n), 1))
    f_ref[:, 1, :] = jnp.dot(oh.astype(_F32), tab_ref[...],
                             preferred_element_type=_F32)
    w = w_ref[...]
    bvec = b_ref[...]
    acc = jnp.zeros((_BB, _D), _F32)
    for r in range(_UMR):
        x = p_ref[:, r, :].astype(_F32)
        h = jnp.dot(x, w, preferred_element_type=_F32) + bvec
        acc = acc + jnp.maximum(h, 0.0)
    f_ref[:, 0, :] = e_ref[...]
    f_ref[:, 2, :] = acc


def _dense_side(p3, W, b2, tab, v2, emb, n):
    return pl.pallas_call(
        functools.partial(_dense_body, n=n),
        grid=(_B // _BB,),
        in_specs=[
            pl.BlockSpec((_BB, _UMR, _D), lambda i: (i, 0, 0)),
            pl.BlockSpec((_D, _D), lambda i: (0, 0)),
            pl.BlockSpec((1, _D), lambda i: (0, 0)),
            pl.BlockSpec((n, _D), lambda i: (0, 0)),
            pl.BlockSpec((_BB, 1), lambda i: (i, 0)),
            pl.BlockSpec((_BB, _D), lambda i: (i, 0)),
        ],
        out_specs=pl.BlockSpec((_BB, 3, _D), lambda i: (i, 0, 0)),
        out_shape=jax.ShapeDtypeStruct((_B, 3, _D), _F32),
    )(p3, W, b2, tab, v2, emb)


def kernel(user_reviews, item_reviews, uids, iids, user_item2id, item_user2id,
           user_item_ratio, item_user_ratio, user_doc, item_doc, type, month,
           user_word_table, item_word_table, user_id_table, item_id_table,
           type_table, month_table, Wu, bu, Wi, bi):
    ue, ie = _idgather(user_id_table, item_id_table, uids, iids)
    ur2 = user_reviews.reshape(_IROWS, _IW)
    uwt_g, ur2 = lax.optimization_barrier((user_word_table, ur2))
    pu3 = _sc_pool(ur2, uwt_g.astype(jnp.bfloat16))
    user_fea = _dense_side(pu3, Wu, bu.reshape(1, _D), type_table,
                           type.astype(jnp.int32).reshape(_B, 1), ue, _TYPE_N)
    ir2 = item_reviews.reshape(_IROWS, _IW)
    iwt_g, ir2 = lax.optimization_barrier((item_word_table, ir2))
    pi3 = _sc_pool(ir2, iwt_g.astype(jnp.bfloat16))
    item_fea = _dense_side(pi3, Wi, bi.reshape(1, _D), month_table,
                           month.astype(jnp.int32).reshape(_B, 1), ie,
                           _MONTH_N)
    return (user_fea, item_fea)


# speedup vs baseline: 1.0566x; 1.0566x over previous
import functools

import jax
import jax.numpy as jnp
import numpy as np
from jax import lax
from jax.experimental import pallas as pl
from jax.experimental.pallas import tpu as pltpu
from jax.experimental.pallas import tpu_sc as plsc

_B = 1024
_UMR = 20
_IMR = 20
_RML = 50
_D = 64
_TYPE_N = 10
_MONTH_N = 12
_NC = 2
_NS = 16
_NW = _NC * _NS
_BW = _B // _NW
_GB = 4
_ROWS = _GB * _RML
_BPB = _UMR // _GB
_NBLK = _BW * _BPB
_NBUF = 4
_IW = 3200
_IROWS = _B * _UMR * _RML // _IW
_IRW = _IROWS // _NW
_BLKR = _IW // _ROWS
_F32 = jnp.float32


def _build_sc_pool():
    mesh = plsc.VectorSubcoreMesh(core_axis_name="c", subcore_axis_name="s")
    bf16 = jnp.bfloat16
    out_type = jax.ShapeDtypeStruct((_B, _UMR, _D), bf16)
    scratch_types = [
        pltpu.VMEM((_IRW, _IW), jnp.int32),
        pltpu.VMEM((_NBUF, _ROWS, _D), bf16),
        pltpu.VMEM((_BW, _UMR, _D), bf16),
        tuple(pltpu.SemaphoreType.DMA for _ in range(_NBUF)),
    ]
    _MSK = np.int32(-65536)

    @functools.partial(
        pl.kernel, out_type=out_type, mesh=mesh, scratch_types=scratch_types,
        compiler_params=pltpu.CompilerParams(
            use_tc_tiling_on_sc=False, needs_layout_passes=False))
    def sc_pool(idx_hbm, tab_hbm,
                out_hbm,
                idx_v, rows_v, pool_v, sems):
        wid = lax.axis_index("s") * _NC + lax.axis_index("c")

        if True:
            pltpu.sync_copy(idx_hbm.at[pl.ds(wid * _IRW, _IRW)], idx_v)

            def start(blk, b):
                row = blk // _BLKR
                o = (blk % _BLKR) * _ROWS
                pltpu.make_async_copy(
                    tab_hbm.at[idx_v.at[row].at[pl.ds(o, _ROWS)]],
                    rows_v.at[b], sems[b]).start()

            def wait(b):
                pltpu.make_async_copy(
                    tab_hbm.at[idx_v.at[0].at[pl.ds(0, _ROWS)]],
                    rows_v.at[b], sems[b]).wait()

            for b in range(_NBUF):
                start(b, b)

            def block_body(t, carry):
                for b in range(_NBUF):
                    blk = _NBUF * t + b
                    wait(b)
                    rows = rows_v.at[b]
                    bat = blk // _BPB
                    r0 = (blk % _BPB) * _GB
                    for s in range(_GB):
                        base = s * _RML

                        def pair(r, rows=rows, base=base):
                            out = []
                            for c in range(2):
                                a = rows[base + r, pl.ds(c * 32, 32)]
                                bb = rows[base + r + 1, pl.ds(c * 32, 32)]
                                lo, hi = plsc.unpack(
                                    a + bb, format=plsc.PackFormat.INTERLEAVED)
                                out += [lo, hi]
                            return tuple(out)

                        acc = pair(0)

                        def row_body(t2, a):
                            p = pair(2 + 2 * t2)
                            return tuple(x + y for x, y in zip(a, p))

                        acc = lax.fori_loop(0, _RML // 2 - 1, row_body, acc)
                        for c in range(2):
                            packed = plsc.pack(
                                acc[2 * c], acc[2 * c + 1],
                                format=plsc.PackFormat.INTERLEAVED)
                            pool_v[bat, r0 + s, pl.ds(c * 32, 32)] = packed
                    nblk = blk + _NBUF

                    @pl.when(nblk < _NBLK)
                    def _():
                        start(nblk, b)
                return carry

            lax.fori_loop(0, _NBLK // _NBUF, block_body, 0)
            pltpu.sync_copy(pool_v, out_hbm.at[pl.ds(wid * _BW, _BW)])

    return sc_pool


_sc_pool = _build_sc_pool()

_BB = 256


def _idgather_body(uids_sm, iids_sm, uidtab_ref, iidtab_ref,
                   ue_ref, ie_ref, sem_u, sem_i):
    i = pl.program_id(0)

    def id_copy(ids_sm, tab_ref, out_ref, sem, j):
        rid = ids_sm[i * _BB + j]
        return pltpu.make_async_copy(
            tab_ref.at[pl.ds(rid, 1)], out_ref.at[pl.ds(j, 1)], sem)

    for j in range(_BB):
        id_copy(uids_sm, uidtab_ref, ue_ref, sem_u, j).start()
        id_copy(iids_sm, iidtab_ref, ie_ref, sem_i, j).start()
    for j in range(_BB):
        id_copy(uids_sm, uidtab_ref, ue_ref, sem_u, j).wait()
        id_copy(iids_sm, iidtab_ref, ie_ref, sem_i, j).wait()


def _idgather(uid_tab, iid_tab, uids, iids):
    grid_spec = pltpu.PrefetchScalarGridSpec(
        num_scalar_prefetch=2,
        grid=(_B // _BB,),
        in_specs=[
            pl.BlockSpec(memory_space=pltpu.HBM),
            pl.BlockSpec(memory_space=pltpu.HBM),
        ],
        out_specs=[
            pl.BlockSpec((_BB, _D), lambda i, *_: (i, 0)),
            pl.BlockSpec((_BB, _D), lambda i, *_: (i, 0)),
        ],
        scratch_shapes=[
            pltpu.SemaphoreType.DMA,
            pltpu.SemaphoreType.DMA,
        ],
    )
    return pl.pallas_call(
        _idgather_body,
        grid_spec=grid_spec,
        out_shape=[
            jax.ShapeDtypeStruct((_B, _D), _F32),
            jax.ShapeDtypeStruct((_B, _D), _F32),
        ],
    )(uids, iids, uid_tab, iid_tab)


def _dense_body(p_ref, w_ref, b_ref, tab_ref, v_ref, e_ref, f_ref, *, n):
    v = v_ref[...]
    oh = (v == lax.broadcasted_iota(jnp.int32, (_BB, n), 1))
    f_ref[:, 1, :] = jnp.dot(oh.astype(_F32), tab_ref[...],
                             preferred_element_type=_F32)
    w = w_ref[...]
    bvec = b_ref[...]
    acc = jnp.zeros((_BB, _D), _F32)
    for r in range(_UMR):
        x = p_ref[:, r, :].astype(_F32)
        h = jnp.dot(x, w, preferred_element_type=_F32) + bvec
        acc = acc + jnp.maximum(h, 0.0)
    f_ref[:, 0, :] = e_ref[...]
    f_ref[:, 2, :] = acc


def _dense_side(p3, W, b2, tab, v2, emb, n):
    return pl.pallas_call(
        functools.partial(_dense_body, n=n),
        grid=(_B // _BB,),
        in_specs=[
            pl.BlockSpec((_BB, _UMR, _D), lambda i: (i, 0, 0)),
            pl.BlockSpec((_D, _D), lambda i: (0, 0)),
            pl.BlockSpec((1, _D), lambda i: (0, 0)),
            pl.BlockSpec((n, _D), lambda i: (0, 0)),
            pl.BlockSpec((_BB, 1), lambda i: (i, 0)),
            pl.BlockSpec((_BB, _D), lambda i: (i, 0)),
        ],
        out_specs=pl.BlockSpec((_BB, 3, _D), lambda i: (i, 0, 0)),
        out_shape=jax.ShapeDtypeStruct((_B, 3, _D), _F32),
    )(p3, W, b2, tab, v2, emb)


def kernel(user_reviews, item_reviews, uids, iids, user_item2id, item_user2id,
           user_item_ratio, item_user_ratio, user_doc, item_doc, type, month,
           user_word_table, item_word_table, user_id_table, item_id_table,
           type_table, month_table, Wu, bu, Wi, bi):
    ue, ie = _idgather(user_id_table, item_id_table, uids, iids)
    pu3 = _sc_pool(user_reviews.reshape(_IROWS, _IW),
                   user_word_table.astype(jnp.bfloat16))
    user_fea = _dense_side(pu3, Wu, bu.reshape(1, _D), type_table,
                           type.astype(jnp.int32).reshape(_B, 1), ue, _TYPE_N)
    pi3 = _sc_pool(item_reviews.reshape(_IROWS, _IW),
                   item_word_table.astype(jnp.bfloat16))
    item_fea = _dense_side(pi3, Wi, bi.reshape(1, _D), month_table,
                           month.astype(jnp.int32).reshape(_B, 1), ie,
                           _MONTH_N)
    return (user_fea, item_fea)


# speedup vs baseline: 1.0651x; 1.0081x over previous
import functools

import jax
import jax.numpy as jnp
import numpy as np
from jax import lax
from jax.experimental import pallas as pl
from jax.experimental.pallas import tpu as pltpu
from jax.experimental.pallas import tpu_sc as plsc

_B = 1024
_UMR = 20
_IMR = 20
_RML = 50
_D = 64
_TYPE_N = 10
_MONTH_N = 12
_NC = 2
_NS = 16
_NW = _NC * _NS
_BW = _B // _NW
_GB = 4
_ROWS = _GB * _RML
_BPB = _UMR // _GB
_NBLK = _BW * _BPB
_NBUF = 5
_IW = 3200
_IROWS = _B * _UMR * _RML // _IW
_IRW = _IROWS // _NW
_BLKR = _IW // _ROWS
_F32 = jnp.float32


def _build_sc_pool():
    mesh = plsc.VectorSubcoreMesh(core_axis_name="c", subcore_axis_name="s")
    bf16 = jnp.bfloat16
    out_type = jax.ShapeDtypeStruct((_B, _UMR, _D), bf16)
    scratch_types = [
        pltpu.VMEM((_IRW, _IW), jnp.int32),
        pltpu.VMEM((_NBUF, _ROWS, _D), bf16),
        pltpu.VMEM((_BW, _UMR, _D), bf16),
        tuple(pltpu.SemaphoreType.DMA for _ in range(_NBUF)),
    ]
    _MSK = np.int32(-65536)

    @functools.partial(
        pl.kernel, out_type=out_type, mesh=mesh, scratch_types=scratch_types,
        compiler_params=pltpu.CompilerParams(
            use_tc_tiling_on_sc=False, needs_layout_passes=False))
    def sc_pool(idx_hbm, tab_hbm,
                out_hbm,
                idx_v, rows_v, pool_v, sems):
        wid = lax.axis_index("s") * _NC + lax.axis_index("c")

        if True:
            pltpu.sync_copy(idx_hbm.at[pl.ds(wid * _IRW, _IRW)], idx_v)

            def start(blk, b):
                row = blk // _BLKR
                o = (blk % _BLKR) * _ROWS
                pltpu.make_async_copy(
                    tab_hbm.at[idx_v.at[row].at[pl.ds(o, _ROWS)]],
                    rows_v.at[b], sems[b]).start()

            def wait(b):
                pltpu.make_async_copy(
                    tab_hbm.at[idx_v.at[0].at[pl.ds(0, _ROWS)]],
                    rows_v.at[b], sems[b]).wait()

            for b in range(_NBUF):
                start(b, b)

            def block_body(t, carry):
                for b in range(_NBUF):
                    blk = _NBUF * t + b
                    wait(b)
                    rows = rows_v.at[b]
                    bat = blk // _BPB
                    r0 = (blk % _BPB) * _GB
                    for s in range(_GB):
                        base = s * _RML

                        def pair(r, rows=rows, base=base):
                            out = []
                            for c in range(2):
                                a = rows[base + r, pl.ds(c * 32, 32)]
                                bb = rows[base + r + 1, pl.ds(c * 32, 32)]
                                lo, hi = plsc.unpack(
                                    a + bb, format=plsc.PackFormat.INTERLEAVED)
                                out += [lo, hi]
                            return tuple(out)

                        acc = pair(0)

                        def row_body(t2, a):
                            p = pair(2 + 2 * t2)
                            return tuple(x + y for x, y in zip(a, p))

                        acc = lax.fori_loop(0, _RML // 2 - 1, row_body, acc)
                        for c in range(2):
                            packed = plsc.pack(
                                acc[2 * c], acc[2 * c + 1],
                                format=plsc.PackFormat.INTERLEAVED)
                            pool_v[bat, r0 + s, pl.ds(c * 32, 32)] = packed
                    nblk = blk + _NBUF

                    @pl.when(nblk < _NBLK)
                    def _():
                        start(nblk, b)
                return carry

            lax.fori_loop(0, _NBLK // _NBUF, block_body, 0)
            pltpu.sync_copy(pool_v, out_hbm.at[pl.ds(wid * _BW, _BW)])

    return sc_pool


_sc_pool = _build_sc_pool()

_BB = 256


def _idgather_body(uids_sm, iids_sm, uidtab_ref, iidtab_ref,
                   ue_ref, ie_ref, sem_u, sem_i):
    i = pl.program_id(0)

    def id_copy(ids_sm, tab_ref, out_ref, sem, j):
        rid = ids_sm[i * _BB + j]
        return pltpu.make_async_copy(
            tab_ref.at[pl.ds(rid, 1)], out_ref.at[pl.ds(j, 1)], sem)

    for j in range(_BB):
        id_copy(uids_sm, uidtab_ref, ue_ref, sem_u, j).start()
        id_copy(iids_sm, iidtab_ref, ie_ref, sem_i, j).start()
    for j in range(_BB):
        id_copy(uids_sm, uidtab_ref, ue_ref, sem_u, j).wait()
        id_copy(iids_sm, iidtab_ref, ie_ref, sem_i, j).wait()


def _idgather(uid_tab, iid_tab, uids, iids):
    grid_spec = pltpu.PrefetchScalarGridSpec(
        num_scalar_prefetch=2,
        grid=(_B // _BB,),
        in_specs=[
            pl.BlockSpec(memory_space=pltpu.HBM),
            pl.BlockSpec(memory_space=pltpu.HBM),
        ],
        out_specs=[
            pl.BlockSpec((_BB, _D), lambda i, *_: (i, 0)),
            pl.BlockSpec((_BB, _D), lambda i, *_: (i, 0)),
        ],
        scratch_shapes=[
            pltpu.SemaphoreType.DMA,
            pltpu.SemaphoreType.DMA,
        ],
    )
    return pl.pallas_call(
        _idgather_body,
        grid_spec=grid_spec,
        out_shape=[
            jax.ShapeDtypeStruct((_B, _D), _F32),
            jax.ShapeDtypeStruct((_B, _D), _F32),
        ],
    )(uids, iids, uid_tab, iid_tab)


def _dense_body(p_ref, w_ref, b_ref, tab_ref, v_ref, e_ref, f_ref, *, n):
    v = v_ref[...]
    oh = (v == lax.broadcasted_iota(jnp.int32, (_BB, n), 1))
    f_ref[:, 1, :] = jnp.dot(oh.astype(_F32), tab_ref[...],
                             preferred_element_type=_F32)
    w = w_ref[...]
    bvec = b_ref[...]
    acc = jnp.zeros((_BB, _D), _F32)
    for r in range(_UMR):
        x = p_ref[:, r, :].astype(_F32)
        h = jnp.dot(x, w, preferred_element_type=_F32) + bvec
        acc = acc + jnp.maximum(h, 0.0)
    f_ref[:, 0, :] = e_ref[...]
    f_ref[:, 2, :] = acc


def _dense_side(p3, W, b2, tab, v2, emb, n):
    return pl.pallas_call(
        functools.partial(_dense_body, n=n),
        grid=(_B // _BB,),
        in_specs=[
            pl.BlockSpec((_BB, _UMR, _D), lambda i: (i, 0, 0)),
            pl.BlockSpec((_D, _D), lambda i: (0, 0)),
            pl.BlockSpec((1, _D), lambda i: (0, 0)),
            pl.BlockSpec((n, _D), lambda i: (0, 0)),
            pl.BlockSpec((_BB, 1), lambda i: (i, 0)),
            pl.BlockSpec((_BB, _D), lambda i: (i, 0)),
        ],
        out_specs=pl.BlockSpec((_BB, 3, _D), lambda i: (i, 0, 0)),
        out_shape=jax.ShapeDtypeStruct((_B, 3, _D), _F32),
    )(p3, W, b2, tab, v2, emb)


def kernel(user_reviews, item_reviews, uids, iids, user_item2id, item_user2id,
           user_item_ratio, item_user_ratio, user_doc, item_doc, type, month,
           user_word_table, item_word_table, user_id_table, item_id_table,
           type_table, month_table, Wu, bu, Wi, bi):
    ue, ie = _idgather(user_id_table, item_id_table, uids, iids)
    pu3 = _sc_pool(user_reviews.reshape(_IROWS, _IW),
                   user_word_table.astype(jnp.bfloat16))
    user_fea = _dense_side(pu3, Wu, bu.reshape(1, _D), type_table,
                           type.astype(jnp.int32).reshape(_B, 1), ue, _TYPE_N)
    pi3 = _sc_pool(item_reviews.reshape(_IROWS, _IW),
                   item_word_table.astype(jnp.bfloat16))
    item_fea = _dense_side(pi3, Wi, bi.reshape(1, _D), month_table,
                           month.astype(jnp.int32).reshape(_B, 1), ie,
                           _MONTH_N)
    return (user_fea, item_fea)
